# Initial kernel scaffold; baseline (speedup 1.0000x reference)
import jax
import jax.numpy as jnp
from jax.experimental import pallas as pl


def kernel(coordinates, W0, b0, R0, W1, b1, R1, W2, b2, R2, clf_w, clf_b):
    raise NotImplementedError("write your pallas kernel here")


# speedup vs baseline: 3.5808x; 3.5808x over previous
import functools
import numpy as np
import jax
import jax.numpy as jnp
from jax import lax
from jax.experimental import pallas as pl
from jax.experimental.pallas import tpu as pltpu

F32 = jnp.float32
I32 = jnp.int32
HI = lax.Precision.HIGHEST
BIG = 1e30
ROWS = 128
CCH = 512

CIN = [352, 64, 128]
FOUT = [64, 128, 256]
LAM = 2.0
THIRD = np.float32(1.0 / 3.0)


def _build_d2(d2_ref, cb, ct, rowio_base, n):
    cch = min(CCH, n)
    rio = rowio_base + lax.broadcasted_iota(I32, (ROWS, cch), 0)
    for j in range(n // cch):
        sl = slice(j * cch, (j + 1) * cch)
        cio = j * cch + lax.broadcasted_iota(I32, (ROWS, cch), 1)
        dx = cb[:, 0:1] - ct[0:1, sl]
        dy = cb[:, 1:2] - ct[1:2, sl]
        dz = cb[:, 2:3] - ct[2:3, sl]
        d2 = dx * dx + dy * dy + dz * dz
        d2_ref[:, sl] = jnp.where(cio == rio, BIG, d2)


def _extract_min(d2_ref, colio, n):
    d2c = d2_ref[...]
    m = jnp.min(d2c, axis=1, keepdims=True)
    am = jnp.min(jnp.where(d2c == m, colio, n), axis=1, keepdims=True)
    oh = colio == am
    d2_ref[...] = jnp.where(oh, BIG, d2c)
    return m, am, oh.astype(F32)


def _normalize_body(c_ref, o_ref):
    c = c_ref[0]
    c = c - jnp.mean(c, axis=0, keepdims=True)
    nrm = jnp.sqrt(jnp.sum(c * c, axis=-1))
    o_ref[0] = c / (jnp.max(nrm) + 1e-8)


def _normalize(coords):
    b, n, _ = coords.shape
    return pl.pallas_call(
        _normalize_body,
        grid=(b,),
        in_specs=[pl.BlockSpec((1, n, 3), lambda i: (i, 0, 0))],
        out_specs=pl.BlockSpec((1, n, 3), lambda i: (i, 0, 0)),
        out_shape=jax.ShapeDtypeStruct((b, n, 3), F32),
    )(coords)


def _shot_body(cf_ref, ct_ref, cb_ref, sig_ref, d2_ref, dx_ref, dy_ref,
               dz_ref, dd_ref, n):
    i = pl.program_id(1)
    cf = cf_ref[0]
    ct = ct_ref[0]
    cb = cb_ref[0]
    colio = lax.broadcasted_iota(I32, (ROWS, n), 1)
    _build_d2(d2_ref, cb, ct, i * ROWS, n)
    kio = lax.broadcasted_iota(I32, (1, 32), 1)

    def step(k, _):
        m, _am, ohf = _extract_min(d2_ref, colio, n)
        nbr = jnp.dot(ohf, cf, precision=HI)
        sel = kio == k
        dx_ref[...] = jnp.where(sel, nbr[:, 0:1] - cb[:, 0:1], dx_ref[...])
        dy_ref[...] = jnp.where(sel, nbr[:, 1:2] - cb[:, 1:2], dy_ref[...])
        dz_ref[...] = jnp.where(sel, nbr[:, 2:3] - cb[:, 2:3], dz_ref[...])
        dd_ref[...] = jnp.where(sel, jnp.sqrt(m + 1e-12), dd_ref[...])
        return 0

    lax.fori_loop(0, 32, step, 0)

    d = dd_ref[...]
    rmax = jnp.max(d, axis=1, keepdims=True) + 1e-8
    x_ = dx_ref[...] + 1e-12
    y = dy_ref[...]
    z = dz_ref[...]
    upper = jnp.where(x_ > 0, jnp.where(y < x_, 4, 5),
                      jnp.where(y > -x_, 6, 7))
    lower = jnp.where(x_ < 0, jnp.where(-y < -x_, 0, 1),
                      jnp.where(-y > x_, 2, 3))
    azb = jnp.where(y >= 0, upper, lower)
    elb = (z / d + 1.0 >= 1.0).astype(I32)
    t = d / rmax
    radb = (t * 2.0 >= 1.0).astype(I32)
    u = t * 11.0
    histb = jnp.zeros_like(azb)
    for j in range(1, 11):
        histb += (u >= j).astype(I32)
    bins = ((azb * 2 + elb) * 2 + radb) * 11 + histb
    binio = lax.broadcasted_iota(I32, (1, 352), 1)
    sig_ref[0] = jnp.zeros((ROWS, 352), F32)
    for k in range(32):
        sig_ref[0] += (bins[:, k : k + 1] == binio).astype(F32)
    sig_ref[0] = sig_ref[0] * np.float32(1.0 / 32.0)


def _shot(c, ct):
    b, n, _ = c.shape
    return pl.pallas_call(
        functools.partial(_shot_body, n=n),
        grid=(b, n // ROWS),
        in_specs=[
            pl.BlockSpec((1, n, 3), lambda bi, i: (bi, 0, 0)),
            pl.BlockSpec((1, 3, n), lambda bi, i: (bi, 0, 0)),
            pl.BlockSpec((1, ROWS, 3), lambda bi, i: (bi, i, 0)),
        ],
        out_specs=pl.BlockSpec((1, ROWS, 352), lambda bi, i: (bi, i, 0)),
        out_shape=jax.ShapeDtypeStruct((b, n, 352), F32),
        scratch_shapes=[
            pltpu.VMEM((ROWS, n), F32),
            pltpu.VMEM((ROWS, 32), F32),
            pltpu.VMEM((ROWS, 32), F32),
            pltpu.VMEM((ROWS, 32), F32),
            pltpu.VMEM((ROWS, 32), F32),
        ],
    )(c, ct, c)


def _bary_body(cf_ref, ct_ref, cb_ref, sig_ref, w_ref, g_ref,
               d2_ref, l_ref, n, radius, gprec):
    i = pl.program_id(1)
    cf = cf_ref[0]
    ct = ct_ref[0]
    cb = cb_ref[0]
    colio = lax.broadcasted_iota(I32, (ROWS, n), 1)
    _build_d2(d2_ref, cb, ct, i * ROWS, n)

    tio = lax.broadcasted_iota(I32, (1, 40), 1)
    rad_i = (tio // 8).astype(F32)
    ang_i = (tio % 8).astype(F32)
    rr2 = np.float32(radius / 5.0) * (rad_i + 1.0)
    ppv = np.float32(2.0 * np.pi / 8.0) * (ang_i + 0.5) - np.float32(np.pi)
    cpp = jnp.cos(ppv)
    spp = jnp.sin(ppv)
    inv_r2 = np.float32(1.0 / (radius * radius))

    def step(k, _):
        m, _am, ohf = _extract_min(d2_ref, colio, n)
        g_ref[0, k] = jnp.dot(ohf, sig_ref[0], precision=gprec)
        nbr = jnp.dot(ohf, cf, precision=HI)
        relx = nbr[:, 0:1] - cb[:, 0:1] + 1e-12
        rely = nbr[:, 1:2] - cb[:, 1:2]
        hyp = jnp.maximum(jnp.sqrt(relx * relx + rely * rely), 1e-30)
        cost = relx / hyp
        sint = rely / hyp
        d = jnp.sqrt(m + 1e-12)
        dr = d - rr2
        logit = -LAM * (dr * dr * inv_r2 + (1.0 - (cost * cpp + sint * spp)))
        l_ref[k] = logit[None]
        return 0

    lax.fori_loop(0, 10, step, 0)

    mx = l_ref[0]
    for k in range(1, 10):
        mx = jnp.maximum(mx, l_ref[k])
    s = jnp.zeros((1, ROWS, 40), F32)
    for k in range(10):
        e = jnp.exp(l_ref[k] - mx)
        s += e
        w_ref[0, k] = e[0]
    inv_s = (1.0 / s)[0]
    for k in range(10):
        w_ref[0, k] = w_ref[0, k] * inv_s


def _bary(c, ct, sig, radius, gprec):
    b, n, _ = c.shape
    cdim = sig.shape[-1]
    return pl.pallas_call(
        functools.partial(_bary_body, n=n, radius=radius, gprec=gprec),
        grid=(b, n // ROWS),
        in_specs=[
            pl.BlockSpec((1, n, 3), lambda bi, i: (bi, 0, 0)),
            pl.BlockSpec((1, 3, n), lambda bi, i: (bi, 0, 0)),
            pl.BlockSpec((1, ROWS, 3), lambda bi, i: (bi, i, 0)),
            pl.BlockSpec((1, n, cdim), lambda bi, i: (bi, 0, 0)),
        ],
        out_specs=[
            pl.BlockSpec((1, 10, ROWS, 40), lambda bi, i: (bi, 0, i, 0)),
            pl.BlockSpec((1, 10, ROWS, cdim), lambda bi, i: (bi, 0, i, 0)),
        ],
        out_shape=[
            jax.ShapeDtypeStruct((b, 10, n, 40), F32),
            jax.ShapeDtypeStruct((b, 10, n, cdim), F32),
        ],
        scratch_shapes=[
            pltpu.VMEM((ROWS, n), F32),
            pltpu.VMEM((10, 1, ROWS, 40), F32),
        ],
    )(c, ct, c, sig)


def _resnet_body(g_ref, w_ref, sig_ref, wt_ref, r_ref, b_ref, out_ref,
                 acc_ref):
    sig = sig_ref[0]
    acc_ref[...] = jnp.dot(sig, r_ref[...], precision=HI) + b_ref[...]
    for t in range(40):
        interp = w_ref[0, 0, :, t : t + 1] * g_ref[0, 0]
        for k in range(1, 10):
            interp += w_ref[0, k, :, t : t + 1] * g_ref[0, k]
        acc_ref[...] += jnp.dot(interp, wt_ref[t], precision=HI)
    out_ref[0] = jnp.maximum(acc_ref[...], 0.0)


def _resnet(g, w, sig, wt, r, bias):
    b, n, cdim = sig.shape
    f = wt.shape[-1]
    return pl.pallas_call(
        _resnet_body,
        grid=(b, n // ROWS),
        in_specs=[
            pl.BlockSpec((1, 10, ROWS, cdim), lambda bi, i: (bi, 0, i, 0)),
            pl.BlockSpec((1, 10, ROWS, 40), lambda bi, i: (bi, 0, i, 0)),
            pl.BlockSpec((1, ROWS, cdim), lambda bi, i: (bi, i, 0)),
            pl.BlockSpec((40, cdim, f), lambda bi, i: (0, 0, 0)),
            pl.BlockSpec((cdim, f), lambda bi, i: (0, 0)),
            pl.BlockSpec((1, f), lambda bi, i: (0, 0)),
        ],
        out_specs=pl.BlockSpec((1, ROWS, f), lambda bi, i: (bi, i, 0)),
        out_shape=jax.ShapeDtypeStruct((b, n, f), F32),
        scratch_shapes=[pltpu.VMEM((ROWS, f), F32)],
    )(g, w, sig, wt, r, bias.reshape(1, f))


def _gravity_body(cf_ref, ct_ref, cb_ref, out_ref, d2_ref, acc_ref, n):
    i = pl.program_id(1)
    cf = cf_ref[0]
    ct = ct_ref[0]
    cb = cb_ref[0]
    colio = lax.broadcasted_iota(I32, (ROWS, n), 1)
    _build_d2(d2_ref, cb, ct, i * ROWS, n)
    acc_ref[...] = jnp.zeros((ROWS, n), F32)

    def step(k, _):
        _m, _am, ohf = _extract_min(d2_ref, colio, n)
        acc_ref[...] += ohf
        return 0

    lax.fori_loop(0, 16, step, 0)
    nm = jnp.dot(acc_ref[...], cf, precision=HI) * np.float32(1.0 / 16.0)
    out_ref[0] = cb + THIRD * (nm - cb)


def _gravity(c, ct):
    b, n, _ = c.shape
    return pl.pallas_call(
        functools.partial(_gravity_body, n=n),
        grid=(b, n // ROWS),
        in_specs=[
            pl.BlockSpec((1, n, 3), lambda bi, i: (bi, 0, 0)),
            pl.BlockSpec((1, 3, n), lambda bi, i: (bi, 0, 0)),
            pl.BlockSpec((1, ROWS, 3), lambda bi, i: (bi, i, 0)),
        ],
        out_specs=pl.BlockSpec((1, ROWS, 3), lambda bi, i: (bi, i, 0)),
        out_shape=jax.ShapeDtypeStruct((b, n, 3), F32),
        scratch_shapes=[
            pltpu.VMEM((ROWS, n), F32),
            pltpu.VMEM((ROWS, n), F32),
        ],
    )(c, ct, c)


def _clf_body(sig_ref, w_ref, b_ref, out_ref):
    msig = jnp.mean(sig_ref[0], axis=0, keepdims=True)
    out_ref[0] = jnp.dot(msig, w_ref[...], precision=HI) + b_ref[...]


def _clf(sig, clf_w, clf_b):
    b, n, cdim = sig.shape
    out = pl.pallas_call(
        _clf_body,
        grid=(b,),
        in_specs=[
            pl.BlockSpec((1, n, cdim), lambda bi: (bi, 0, 0)),
            pl.BlockSpec((cdim, 40), lambda bi: (0, 0)),
            pl.BlockSpec((1, 40), lambda bi: (0, 0)),
        ],
        out_specs=pl.BlockSpec((1, 1, 40), lambda bi: (bi, 0, 0)),
        out_shape=jax.ShapeDtypeStruct((b, 1, 40), F32),
    )(sig, clf_w, clf_b.reshape(1, 40))
    return out.reshape(b, 40)


def kernel(coordinates, W0, b0, R0, W1, b1, R1, W2, b2, R2, clf_w, clf_b):
    Ws = [W0, W1, W2]
    bs = [b0, b1, b2]
    Rs = [R0, R1, R2]
    c = _normalize(coordinates)
    ct = c.transpose(0, 2, 1)
    sig = _shot(c, ct)
    for i in range(3):
        radius = 0.05 * (1.0 + 0.25 * i)
        gprec = lax.Precision.DEFAULT if i == 0 else HI
        w, g = _bary(c, ct, sig, radius, gprec)
        sig = _resnet(g, w, sig, Ws[i], Rs[i], bs[i])
        for _ in range(3):
            c = _gravity(c, ct)
            ct = c.transpose(0, 2, 1)
        c = c[:, ::2]
        ct = ct[:, :, ::2]
        sig = sig[:, ::2]
    return _clf(sig, clf_w, clf_b)


# speedup vs baseline: 4.1985x; 1.1725x over previous
import functools
import numpy as np
import jax
import jax.numpy as jnp
from jax import lax
from jax.experimental import pallas as pl
from jax.experimental.pallas import tpu as pltpu

F32 = jnp.float32
I32 = jnp.int32
HI = lax.Precision.HIGHEST
BIG = 1e30
ROWS = 128
CCH = 512

CIN = [352, 64, 128]
FOUT = [64, 128, 256]
LAM = 2.0
THIRD = np.float32(1.0 / 3.0)


def _build_d2(d2_ref, cb, ct, rowio_base, n):
    cch = min(CCH, n)
    rio = rowio_base + lax.broadcasted_iota(I32, (ROWS, cch), 0)
    for j in range(n // cch):
        sl = slice(j * cch, (j + 1) * cch)
        cio = j * cch + lax.broadcasted_iota(I32, (ROWS, cch), 1)
        dx = cb[:, 0:1] - ct[0:1, sl]
        dy = cb[:, 1:2] - ct[1:2, sl]
        dz = cb[:, 2:3] - ct[2:3, sl]
        d2 = dx * dx + dy * dy + dz * dz
        d2_ref[:, sl] = jnp.where(cio == rio, BIG, d2)


def _extract_min(d2_ref, colio, n):
    d2c = d2_ref[...]
    m = jnp.min(d2c, axis=1, keepdims=True)
    am = jnp.min(jnp.where(d2c == m, colio, n), axis=1, keepdims=True)
    oh = colio == am
    d2_ref[...] = jnp.where(oh, BIG, d2c)
    return m, oh


def _split3cat(x):
    h = x.astype(jnp.bfloat16)
    r = x - h.astype(F32)
    mid = r.astype(jnp.bfloat16)
    lo = (r - mid.astype(F32)).astype(jnp.bfloat16)
    return jnp.concatenate([h, mid, lo], axis=-1)


def _oh_dot3(ohb, parts, c):
    p = jnp.dot(ohb, parts, preferred_element_type=F32)
    return p[:, :c] + p[:, c : 2 * c] + p[:, 2 * c :]


def _normalize_body(c_ref, o_ref):
    c = c_ref[0]
    c = c - jnp.mean(c, axis=0, keepdims=True)
    nrm = jnp.sqrt(jnp.sum(c * c, axis=-1))
    o_ref[0] = c / (jnp.max(nrm) + 1e-8)


def _normalize(coords):
    b, n, _ = coords.shape
    return pl.pallas_call(
        _normalize_body,
        grid=(b,),
        in_specs=[pl.BlockSpec((1, n, 3), lambda i: (i, 0, 0))],
        out_specs=pl.BlockSpec((1, n, 3), lambda i: (i, 0, 0)),
        out_shape=jax.ShapeDtypeStruct((b, n, 3), F32),
    )(coords)


def _shot_body(c3_ref, ct_ref, cb_ref, sig_ref, d2_ref, dx_ref, dy_ref,
               dz_ref, dd_ref, n):
    i = pl.program_id(1)
    c3 = c3_ref[0]
    ct = ct_ref[0]
    cb = cb_ref[0]
    colio = lax.broadcasted_iota(I32, (ROWS, n), 1)
    _build_d2(d2_ref, cb, ct, i * ROWS, n)
    kio = lax.broadcasted_iota(I32, (1, 32), 1)

    def step(k, _):
        m, oh = _extract_min(d2_ref, colio, n)
        nbr = _oh_dot3(oh.astype(jnp.bfloat16), c3, 3)
        sel = kio == k
        dx_ref[...] = jnp.where(sel, nbr[:, 0:1] - cb[:, 0:1], dx_ref[...])
        dy_ref[...] = jnp.where(sel, nbr[:, 1:2] - cb[:, 1:2], dy_ref[...])
        dz_ref[...] = jnp.where(sel, nbr[:, 2:3] - cb[:, 2:3], dz_ref[...])
        dd_ref[...] = jnp.where(sel, jnp.sqrt(m + 1e-12), dd_ref[...])
        return 0

    lax.fori_loop(0, 32, step, 0)

    d = dd_ref[...]
    rmax = jnp.max(d, axis=1, keepdims=True) + 1e-8
    x_ = dx_ref[...] + 1e-12
    y = dy_ref[...]
    z = dz_ref[...]
    upper = jnp.where(x_ > 0, jnp.where(y < x_, 4, 5),
                      jnp.where(y > -x_, 6, 7))
    lower = jnp.where(x_ < 0, jnp.where(-y < -x_, 0, 1),
                      jnp.where(-y > x_, 2, 3))
    azb = jnp.where(y >= 0, upper, lower)
    elb = (z / d + 1.0 >= 1.0).astype(I32)
    t = d / rmax
    radb = (t * 2.0 >= 1.0).astype(I32)
    u = t * 11.0
    histb = jnp.zeros_like(azb)
    for j in range(1, 11):
        histb += (u >= j).astype(I32)
    bins = ((azb * 2 + elb) * 2 + radb) * 11 + histb
    binio = lax.broadcasted_iota(I32, (1, 352), 1)
    sig_ref[0] = jnp.zeros((ROWS, 352), F32)
    for k in range(32):
        sig_ref[0] += (bins[:, k : k + 1] == binio).astype(F32)
    sig_ref[0] = sig_ref[0] * np.float32(1.0 / 32.0)


def _shot(c, c3, ct):
    b, n, _ = c.shape
    return pl.pallas_call(
        functools.partial(_shot_body, n=n),
        grid=(b, n // ROWS),
        in_specs=[
            pl.BlockSpec((1, n, 9), lambda bi, i: (bi, 0, 0)),
            pl.BlockSpec((1, 3, n), lambda bi, i: (bi, 0, 0)),
            pl.BlockSpec((1, ROWS, 3), lambda bi, i: (bi, i, 0)),
        ],
        out_specs=pl.BlockSpec((1, ROWS, 352), lambda bi, i: (bi, i, 0)),
        out_shape=jax.ShapeDtypeStruct((b, n, 352), F32),
        scratch_shapes=[
            pltpu.VMEM((ROWS, n), F32),
            pltpu.VMEM((ROWS, 32), F32),
            pltpu.VMEM((ROWS, 32), F32),
            pltpu.VMEM((ROWS, 32), F32),
            pltpu.VMEM((ROWS, 32), F32),
        ],
    )(c3, ct, c)


def _bary_body(cf_ref, ct_ref, cb_ref, sig_ref, w_ref, g_ref,
               d2_ref, l_ref, n, radius, gprec):
    i = pl.program_id(1)
    cf = cf_ref[0]
    ct = ct_ref[0]
    cb = cb_ref[0]
    colio = lax.broadcasted_iota(I32, (ROWS, n), 1)
    _build_d2(d2_ref, cb, ct, i * ROWS, n)

    tio = lax.broadcasted_iota(I32, (1, 40), 1)
    rad_i = (tio // 8).astype(F32)
    ang_i = (tio % 8).astype(F32)
    rr2 = np.float32(radius / 5.0) * (rad_i + 1.0)
    ppv = np.float32(2.0 * np.pi / 8.0) * (ang_i + 0.5) - np.float32(np.pi)
    cpp = jnp.cos(ppv)
    spp = jnp.sin(ppv)
    inv_r2 = np.float32(1.0 / (radius * radius))

    def step(k, _):
        m, oh = _extract_min(d2_ref, colio, n)
        ohf = oh.astype(F32)
        g_ref[0, k] = jnp.dot(ohf, sig_ref[0], precision=gprec)
        nbr = jnp.dot(ohf, cf, precision=HI)
        relx = nbr[:, 0:1] - cb[:, 0:1] + 1e-12
        rely = nbr[:, 1:2] - cb[:, 1:2]
        hyp = jnp.maximum(jnp.sqrt(relx * relx + rely * rely), 1e-30)
        cost = relx / hyp
        sint = rely / hyp
        d = jnp.sqrt(m + 1e-12)
        dr = d - rr2
        logit = -LAM * (dr * dr * inv_r2 + (1.0 - (cost * cpp + sint * spp)))
        l_ref[k] = logit[None]
        return 0

    lax.fori_loop(0, 10, step, 0)

    mx = l_ref[0]
    for k in range(1, 10):
        mx = jnp.maximum(mx, l_ref[k])
    s = jnp.zeros((1, ROWS, 40), F32)
    for k in range(10):
        e = jnp.exp(l_ref[k] - mx)
        s += e
        w_ref[0, k] = e[0]
    inv_s = (1.0 / s)[0]
    for k in range(10):
        w_ref[0, k] = w_ref[0, k] * inv_s


def _bary(c, ct, sig, radius, gprec):
    b, n, _ = c.shape
    cdim = sig.shape[-1]
    return pl.pallas_call(
        functools.partial(_bary_body, n=n, radius=radius, gprec=gprec),
        grid=(b, n // ROWS),
        in_specs=[
            pl.BlockSpec((1, n, 3), lambda bi, i: (bi, 0, 0)),
            pl.BlockSpec((1, 3, n), lambda bi, i: (bi, 0, 0)),
            pl.BlockSpec((1, ROWS, 3), lambda bi, i: (bi, i, 0)),
            pl.BlockSpec((1, n, cdim), lambda bi, i: (bi, 0, 0)),
        ],
        out_specs=[
            pl.BlockSpec((1, 10, ROWS, 40), lambda bi, i: (bi, 0, i, 0)),
            pl.BlockSpec((1, 10, ROWS, cdim), lambda bi, i: (bi, 0, i, 0)),
        ],
        out_shape=[
            jax.ShapeDtypeStruct((b, 10, n, 40), F32),
            jax.ShapeDtypeStruct((b, 10, n, cdim), F32),
        ],
        scratch_shapes=[
            pltpu.VMEM((ROWS, n), F32),
            pltpu.VMEM((10, 1, ROWS, 40), F32),
        ],
    )(c, ct, c, sig)


def _resnet_body(g_ref, w_ref, sig_ref, wt_ref, r_ref, b_ref, out_ref,
                 acc_ref):
    sig = sig_ref[0]
    acc_ref[...] = jnp.dot(sig, r_ref[...], precision=HI) + b_ref[...]
    for t in range(40):
        interp = w_ref[0, 0, :, t : t + 1] * g_ref[0, 0]
        for k in range(1, 10):
            interp += w_ref[0, k, :, t : t + 1] * g_ref[0, k]
        acc_ref[...] += jnp.dot(interp, wt_ref[t], precision=HI)
    out_ref[0] = jnp.maximum(acc_ref[...], 0.0)


def _resnet(g, w, sig, wt, r, bias):
    b, n, cdim = sig.shape
    f = wt.shape[-1]
    return pl.pallas_call(
        _resnet_body,
        grid=(b, n // ROWS),
        in_specs=[
            pl.BlockSpec((1, 10, ROWS, cdim), lambda bi, i: (bi, 0, i, 0)),
            pl.BlockSpec((1, 10, ROWS, 40), lambda bi, i: (bi, 0, i, 0)),
            pl.BlockSpec((1, ROWS, cdim), lambda bi, i: (bi, i, 0)),
            pl.BlockSpec((40, cdim, f), lambda bi, i: (0, 0, 0)),
            pl.BlockSpec((cdim, f), lambda bi, i: (0, 0)),
            pl.BlockSpec((1, f), lambda bi, i: (0, 0)),
        ],
        out_specs=pl.BlockSpec((1, ROWS, f), lambda bi, i: (bi, i, 0)),
        out_shape=jax.ShapeDtypeStruct((b, n, f), F32),
        scratch_shapes=[pltpu.VMEM((ROWS, f), F32)],
    )(g, w, sig, wt, r, bias.reshape(1, f))


def _gravity_body(cf_ref, ct_ref, cb_ref, out_ref, d2_ref, acc_ref, n):
    i = pl.program_id(1)
    cf = cf_ref[0]
    ct = ct_ref[0]
    cb = cb_ref[0]
    colio = lax.broadcasted_iota(I32, (ROWS, n), 1)
    _build_d2(d2_ref, cb, ct, i * ROWS, n)
    acc_ref[...] = jnp.zeros((ROWS, n), F32)

    def step(k, _):
        _m, oh = _extract_min(d2_ref, colio, n)
        acc_ref[...] += oh.astype(F32)
        return 0

    lax.fori_loop(0, 16, step, 0)
    nm = jnp.dot(acc_ref[...], cf, precision=HI) * np.float32(1.0 / 16.0)
    out_ref[0] = cb + THIRD * (nm - cb)


def _gravity(c, ct):
    b, n, _ = c.shape
    return pl.pallas_call(
        functools.partial(_gravity_body, n=n),
        grid=(b, n // ROWS),
        in_specs=[
            pl.BlockSpec((1, n, 3), lambda bi, i: (bi, 0, 0)),
            pl.BlockSpec((1, 3, n), lambda bi, i: (bi, 0, 0)),
            pl.BlockSpec((1, ROWS, 3), lambda bi, i: (bi, i, 0)),
        ],
        out_specs=pl.BlockSpec((1, ROWS, 3), lambda bi, i: (bi, i, 0)),
        out_shape=jax.ShapeDtypeStruct((b, n, 3), F32),
        scratch_shapes=[
            pltpu.VMEM((ROWS, n), F32),
            pltpu.VMEM((ROWS, n), F32),
        ],
    )(c, ct, c)


def _clf_body(sig_ref, w_ref, b_ref, out_ref):
    msig = jnp.mean(sig_ref[0], axis=0, keepdims=True)
    out_ref[0] = jnp.dot(msig, w_ref[...], precision=HI) + b_ref[...]


def _clf(sig, clf_w, clf_b):
    b, n, cdim = sig.shape
    out = pl.pallas_call(
        _clf_body,
        grid=(b,),
        in_specs=[
            pl.BlockSpec((1, n, cdim), lambda bi: (bi, 0, 0)),
            pl.BlockSpec((cdim, 40), lambda bi: (0, 0)),
            pl.BlockSpec((1, 40), lambda bi: (0, 0)),
        ],
        out_specs=pl.BlockSpec((1, 1, 40), lambda bi: (bi, 0, 0)),
        out_shape=jax.ShapeDtypeStruct((b, 1, 40), F32),
    )(sig, clf_w, clf_b.reshape(1, 40))
    return out.reshape(b, 40)


def kernel(coordinates, W0, b0, R0, W1, b1, R1, W2, b2, R2, clf_w, clf_b):
    Ws = [W0, W1, W2]
    bs = [b0, b1, b2]
    Rs = [R0, R1, R2]
    c = _normalize(coordinates)
    ct = c.transpose(0, 2, 1)
    sig = _shot(c, _split3cat(c), ct)
    for i in range(3):
        radius = 0.05 * (1.0 + 0.25 * i)
        gprec = lax.Precision.DEFAULT if i == 0 else HI
        w, g = _bary(c, ct, sig, radius, gprec)
        sig = _resnet(g, w, sig, Ws[i], Rs[i], bs[i])
        for _ in range(3):
            c = _gravity(c, ct)
            ct = c.transpose(0, 2, 1)
        c = c[:, ::2]
        ct = ct[:, :, ::2]
        sig = sig[:, ::2]
    return _clf(sig, clf_w, clf_b)


# speedup vs baseline: 4.5573x; 1.0855x over previous
import functools
import numpy as np
import jax
import jax.numpy as jnp
from jax import lax
from jax.experimental import pallas as pl
from jax.experimental.pallas import tpu as pltpu

F32 = jnp.float32
I32 = jnp.int32
HI = lax.Precision.HIGHEST
BIG = 1e30
ROWS = 128
CCH = 512

CIN = [352, 64, 128]
FOUT = [64, 128, 256]
LAM = 2.0
THIRD = np.float32(1.0 / 3.0)


def _build_d2(d2_ref, cb, ct, rowio_base, n):
    cch = min(CCH, n)
    rio = rowio_base + lax.broadcasted_iota(I32, (ROWS, cch), 0)
    for j in range(n // cch):
        sl = slice(j * cch, (j + 1) * cch)
        cio = j * cch + lax.broadcasted_iota(I32, (ROWS, cch), 1)
        dx = cb[:, 0:1] - ct[0:1, sl]
        dy = cb[:, 1:2] - ct[1:2, sl]
        dz = cb[:, 2:3] - ct[2:3, sl]
        d2 = dx * dx + dy * dy + dz * dz
        d2_ref[:, sl] = jnp.where(cio == rio, BIG, d2)


def _extract_min(d2_ref, colio, n):
    d2c = d2_ref[...]
    m = jnp.min(d2c, axis=1, keepdims=True)
    am = jnp.min(jnp.where(d2c == m, colio, n), axis=1, keepdims=True)
    oh = colio == am
    d2_ref[...] = jnp.where(oh, BIG, d2c)
    return m, oh


def _split3cat(x):
    h = x.astype(jnp.bfloat16)
    r = x - h.astype(F32)
    mid = r.astype(jnp.bfloat16)
    lo = (r - mid.astype(F32)).astype(jnp.bfloat16)
    return jnp.concatenate([h, mid, lo], axis=-1)


def _oh_dot3(ohb, parts, c):
    p = jnp.dot(ohb, parts, preferred_element_type=F32)
    return p[:, :c] + p[:, c : 2 * c] + p[:, 2 * c :]


def _normalize_body(c_ref, o_ref):
    c = c_ref[0]
    c = c - jnp.mean(c, axis=0, keepdims=True)
    nrm = jnp.sqrt(jnp.sum(c * c, axis=-1))
    o_ref[0] = c / (jnp.max(nrm) + 1e-8)


def _normalize(coords):
    b, n, _ = coords.shape
    return pl.pallas_call(
        _normalize_body,
        grid=(b,),
        in_specs=[pl.BlockSpec((1, n, 3), lambda i: (i, 0, 0))],
        out_specs=pl.BlockSpec((1, n, 3), lambda i: (i, 0, 0)),
        out_shape=jax.ShapeDtypeStruct((b, n, 3), F32),
    )(coords)


def _shot_body(c3_ref, ct_ref, cb_ref, sig_ref, d2_ref, dx_ref, dy_ref,
               dz_ref, dd_ref, n):
    i = pl.program_id(1)
    c3 = c3_ref[0]
    ct = ct_ref[0]
    cb = cb_ref[0]
    colio = lax.broadcasted_iota(I32, (ROWS, n), 1)
    _build_d2(d2_ref, cb, ct, i * ROWS, n)
    kio = lax.broadcasted_iota(I32, (1, 32), 1)

    def step(k, _):
        m, oh = _extract_min(d2_ref, colio, n)
        nbr = _oh_dot3(oh.astype(jnp.bfloat16), c3, 3)
        sel = kio == k
        dx_ref[...] = jnp.where(sel, nbr[:, 0:1] - cb[:, 0:1], dx_ref[...])
        dy_ref[...] = jnp.where(sel, nbr[:, 1:2] - cb[:, 1:2], dy_ref[...])
        dz_ref[...] = jnp.where(sel, nbr[:, 2:3] - cb[:, 2:3], dz_ref[...])
        dd_ref[...] = jnp.where(sel, jnp.sqrt(m + 1e-12), dd_ref[...])
        return 0

    lax.fori_loop(0, 32, step, 0)

    d = dd_ref[...]
    rmax = jnp.max(d, axis=1, keepdims=True) + 1e-8
    x_ = dx_ref[...] + 1e-12
    y = dy_ref[...]
    z = dz_ref[...]
    upper = jnp.where(x_ > 0, jnp.where(y < x_, 4, 5),
                      jnp.where(y > -x_, 6, 7))
    lower = jnp.where(x_ < 0, jnp.where(-y < -x_, 0, 1),
                      jnp.where(-y > x_, 2, 3))
    azb = jnp.where(y >= 0, upper, lower)
    elb = (z / d + 1.0 >= 1.0).astype(I32)
    t = d / rmax
    radb = (t * 2.0 >= 1.0).astype(I32)
    u = t * 11.0
    histb = jnp.zeros_like(azb)
    for j in range(1, 11):
        histb += (u >= j).astype(I32)
    bins = ((azb * 2 + elb) * 2 + radb) * 11 + histb
    binio = lax.broadcasted_iota(I32, (1, 352), 1)
    sig_ref[0] = jnp.zeros((ROWS, 352), F32)
    for k in range(32):
        sig_ref[0] += (bins[:, k : k + 1] == binio).astype(F32)
    sig_ref[0] = sig_ref[0] * np.float32(1.0 / 32.0)


def _shot(c, c3, ct):
    b, n, _ = c.shape
    return pl.pallas_call(
        functools.partial(_shot_body, n=n),
        grid=(b, n // ROWS),
        in_specs=[
            pl.BlockSpec((1, n, 9), lambda bi, i: (bi, 0, 0)),
            pl.BlockSpec((1, 3, n), lambda bi, i: (bi, 0, 0)),
            pl.BlockSpec((1, ROWS, 3), lambda bi, i: (bi, i, 0)),
        ],
        out_specs=pl.BlockSpec((1, ROWS, 352), lambda bi, i: (bi, i, 0)),
        out_shape=jax.ShapeDtypeStruct((b, n, 352), F32),
        scratch_shapes=[
            pltpu.VMEM((ROWS, n), F32),
            pltpu.VMEM((ROWS, 32), F32),
            pltpu.VMEM((ROWS, 32), F32),
            pltpu.VMEM((ROWS, 32), F32),
            pltpu.VMEM((ROWS, 32), F32),
        ],
    )(c3, ct, c)


def _bary_body(c3_ref, ct_ref, cb_ref, sig_ref, w_ref, g_ref,
               d2_ref, l_ref, n, radius, cdim, stage0):
    i = pl.program_id(1)
    c3 = c3_ref[0]
    ct = ct_ref[0]
    cb = cb_ref[0]
    colio = lax.broadcasted_iota(I32, (ROWS, n), 1)
    _build_d2(d2_ref, cb, ct, i * ROWS, n)

    tio = lax.broadcasted_iota(I32, (1, 40), 1)
    rad_i = (tio // 8).astype(F32)
    ang_i = (tio % 8).astype(F32)
    rr2 = np.float32(radius / 5.0) * (rad_i + 1.0)
    ppv = np.float32(2.0 * np.pi / 8.0) * (ang_i + 0.5) - np.float32(np.pi)
    cpp = jnp.cos(ppv)
    spp = jnp.sin(ppv)
    inv_r2 = np.float32(1.0 / (radius * radius))

    def step(k, _):
        m, oh = _extract_min(d2_ref, colio, n)
        ohb = oh.astype(jnp.bfloat16)
        if stage0:
            g_ref[0, k] = jnp.dot(oh.astype(F32), sig_ref[0],
                                  precision=lax.Precision.DEFAULT)
        else:
            g_ref[0, k] = _oh_dot3(ohb, sig_ref[0], cdim)
        nbr = _oh_dot3(ohb, c3, 3)
        relx = nbr[:, 0:1] - cb[:, 0:1] + 1e-12
        rely = nbr[:, 1:2] - cb[:, 1:2]
        hyp = jnp.maximum(jnp.sqrt(relx * relx + rely * rely), 1e-30)
        cost = relx / hyp
        sint = rely / hyp
        d = jnp.sqrt(m + 1e-12)
        dr = d - rr2
        logit = -LAM * (dr * dr * inv_r2 + (1.0 - (cost * cpp + sint * spp)))
        l_ref[k] = logit[None]
        return 0

    lax.fori_loop(0, 10, step, 0)

    mx = l_ref[0]
    for k in range(1, 10):
        mx = jnp.maximum(mx, l_ref[k])
    s = jnp.zeros((1, ROWS, 40), F32)
    for k in range(10):
        e = jnp.exp(l_ref[k] - mx)
        s += e
        w_ref[0, k] = e[0]
    inv_s = (1.0 / s)[0]
    for k in range(10):
        w_ref[0, k] = w_ref[0, k] * inv_s


def _bary(c3, ct, cb, sig, radius, cdim, stage0):
    b, n, _ = cb.shape
    sdim = sig.shape[-1]
    return pl.pallas_call(
        functools.partial(_bary_body, n=n, radius=radius, cdim=cdim,
                          stage0=stage0),
        grid=(b, n // ROWS),
        in_specs=[
            pl.BlockSpec((1, n, 9), lambda bi, i: (bi, 0, 0)),
            pl.BlockSpec((1, 3, n), lambda bi, i: (bi, 0, 0)),
            pl.BlockSpec((1, ROWS, 3), lambda bi, i: (bi, i, 0)),
            pl.BlockSpec((1, n, sdim), lambda bi, i: (bi, 0, 0)),
        ],
        out_specs=[
            pl.BlockSpec((1, 10, ROWS, 40), lambda bi, i: (bi, 0, i, 0)),
            pl.BlockSpec((1, 10, ROWS, cdim), lambda bi, i: (bi, 0, i, 0)),
        ],
        out_shape=[
            jax.ShapeDtypeStruct((b, 10, n, 40), F32),
            jax.ShapeDtypeStruct((b, 10, n, cdim), F32),
        ],
        scratch_shapes=[
            pltpu.VMEM((ROWS, n), F32),
            pltpu.VMEM((10, 1, ROWS, 40), F32),
        ],
    )(c3, ct, cb, sig)


def _resnet_body(g_ref, w_ref, sig_ref, wt_ref, r_ref, b_ref, out_ref,
                 o3_ref, acc_ref):
    sig = sig_ref[0]
    acc_ref[...] = jnp.dot(sig, r_ref[...], precision=HI) + b_ref[...]
    for t in range(40):
        interp = w_ref[0, 0, :, t : t + 1] * g_ref[0, 0]
        for k in range(1, 10):
            interp += w_ref[0, k, :, t : t + 1] * g_ref[0, k]
        acc_ref[...] += jnp.dot(interp, wt_ref[t], precision=HI)
    relu = jnp.maximum(acc_ref[...], 0.0)
    out_ref[0] = relu
    o3_ref[0] = _split3cat(relu)


def _resnet(g, w, sig, wt, r, bias):
    b, n, cdim = sig.shape
    f = wt.shape[-1]
    return pl.pallas_call(
        _resnet_body,
        grid=(b, n // ROWS),
        in_specs=[
            pl.BlockSpec((1, 10, ROWS, cdim), lambda bi, i: (bi, 0, i, 0)),
            pl.BlockSpec((1, 10, ROWS, 40), lambda bi, i: (bi, 0, i, 0)),
            pl.BlockSpec((1, ROWS, cdim), lambda bi, i: (bi, i, 0)),
            pl.BlockSpec((40, cdim, f), lambda bi, i: (0, 0, 0)),
            pl.BlockSpec((cdim, f), lambda bi, i: (0, 0)),
            pl.BlockSpec((1, f), lambda bi, i: (0, 0)),
        ],
        out_specs=[
            pl.BlockSpec((1, ROWS, f), lambda bi, i: (bi, i, 0)),
            pl.BlockSpec((1, ROWS, 3 * f), lambda bi, i: (bi, i, 0)),
        ],
        out_shape=[
            jax.ShapeDtypeStruct((b, n, f), F32),
            jax.ShapeDtypeStruct((b, n, 3 * f), jnp.bfloat16),
        ],
        scratch_shapes=[pltpu.VMEM((ROWS, f), F32)],
    )(g, w, sig, wt, r, bias.reshape(1, f))


def _gravity_body(cf_ref, ct_ref, cb_ref, out_ref, d2_ref, acc_ref, n):
    i = pl.program_id(1)
    cf = cf_ref[0]
    ct = ct_ref[0]
    cb = cb_ref[0]
    colio = lax.broadcasted_iota(I32, (ROWS, n), 1)
    _build_d2(d2_ref, cb, ct, i * ROWS, n)
    acc_ref[...] = jnp.zeros((ROWS, n), F32)

    def step(k, _):
        _m, oh = _extract_min(d2_ref, colio, n)
        acc_ref[...] += oh.astype(F32)
        return 0

    lax.fori_loop(0, 16, step, 0)
    nm = jnp.dot(acc_ref[...], cf, precision=HI) * np.float32(1.0 / 16.0)
    out_ref[0] = cb + THIRD * (nm - cb)


def _gravity(c, ct):
    b, n, _ = c.shape
    return pl.pallas_call(
        functools.partial(_gravity_body, n=n),
        grid=(b, n // ROWS),
        in_specs=[
            pl.BlockSpec((1, n, 3), lambda bi, i: (bi, 0, 0)),
            pl.BlockSpec((1, 3, n), lambda bi, i: (bi, 0, 0)),
            pl.BlockSpec((1, ROWS, 3), lambda bi, i: (bi, i, 0)),
        ],
        out_specs=pl.BlockSpec((1, ROWS, 3), lambda bi, i: (bi, i, 0)),
        out_shape=jax.ShapeDtypeStruct((b, n, 3), F32),
        scratch_shapes=[
            pltpu.VMEM((ROWS, n), F32),
            pltpu.VMEM((ROWS, n), F32),
        ],
    )(c, ct, c)


def _clf_body(sig_ref, w_ref, b_ref, out_ref):
    msig = jnp.mean(sig_ref[0], axis=0, keepdims=True)
    out_ref[0] = jnp.dot(msig, w_ref[...], precision=HI) + b_ref[...]


def _clf(sig, clf_w, clf_b):
    b, n, cdim = sig.shape
    out = pl.pallas_call(
        _clf_body,
        grid=(b,),
        in_specs=[
            pl.BlockSpec((1, n, cdim), lambda bi: (bi, 0, 0)),
            pl.BlockSpec((cdim, 40), lambda bi: (0, 0)),
            pl.BlockSpec((1, 40), lambda bi: (0, 0)),
        ],
        out_specs=pl.BlockSpec((1, 1, 40), lambda bi: (bi, 0, 0)),
        out_shape=jax.ShapeDtypeStruct((b, 1, 40), F32),
    )(sig, clf_w, clf_b.reshape(1, 40))
    return out.reshape(b, 40)


def kernel(coordinates, W0, b0, R0, W1, b1, R1, W2, b2, R2, clf_w, clf_b):
    Ws = [W0, W1, W2]
    bs = [b0, b1, b2]
    Rs = [R0, R1, R2]
    c = _normalize(coordinates)
    ct = c.transpose(0, 2, 1)
    c3 = _split3cat(c)
    sig = _shot(c, c3, ct)
    sig3 = None
    for i in range(3):
        radius = 0.05 * (1.0 + 0.25 * i)
        w, g = _bary(c3, ct, c, sig if i == 0 else sig3, radius,
                     CIN[i], stage0=(i == 0))
        sig, sig3 = _resnet(g, w, sig, Ws[i], Rs[i], bs[i])
        for _ in range(3):
            c = _gravity(c, ct)
            ct = c.transpose(0, 2, 1)
        c = c[:, ::2]
        ct = ct[:, :, ::2]
        c3 = _split3cat(c)
        sig = sig[:, ::2]
        sig3 = sig3[:, ::2]
    return _clf(sig, clf_w, clf_b)


# speedup vs baseline: 5.1045x; 1.1201x over previous
import functools
import numpy as np
import jax
import jax.numpy as jnp
from jax import lax
from jax.experimental import pallas as pl
from jax.experimental.pallas import tpu as pltpu

F32 = jnp.float32
I32 = jnp.int32
HI = lax.Precision.HIGHEST
BIG = 1e30
ROWS = 128
CCH = 512

CIN = [352, 64, 128]
FOUT = [64, 128, 256]
LAM = 2.0
THIRD = np.float32(1.0 / 3.0)


def _build_d2(d2_ref, cb, ct, rowio_base, n, rstride=1):
    cch = min(CCH, n)
    rio = (rowio_base + lax.broadcasted_iota(I32, (ROWS, cch), 0)) * rstride
    for j in range(n // cch):
        sl = slice(j * cch, (j + 1) * cch)
        cio = j * cch + lax.broadcasted_iota(I32, (ROWS, cch), 1)
        dx = cb[:, 0:1] - ct[0:1, sl]
        dy = cb[:, 1:2] - ct[1:2, sl]
        dz = cb[:, 2:3] - ct[2:3, sl]
        d2 = dx * dx + dy * dy + dz * dz
        d2_ref[:, sl] = jnp.where(cio == rio, BIG, d2)


def _extract_min(d2_ref, colio, n):
    d2c = d2_ref[...]
    m = jnp.min(d2c, axis=1, keepdims=True)
    am = jnp.min(jnp.where(d2c == m, colio, n), axis=1, keepdims=True)
    oh = colio == am
    d2_ref[...] = jnp.where(oh, BIG, d2c)
    return m, oh


def _split3cat(x):
    h = x.astype(jnp.bfloat16)
    r = x - h.astype(F32)
    mid = r.astype(jnp.bfloat16)
    lo = (r - mid.astype(F32)).astype(jnp.bfloat16)
    return jnp.concatenate([h, mid, lo], axis=-1)


def _oh_dot3(ohb, parts, c):
    p = jnp.dot(ohb, parts, preferred_element_type=F32)
    return p[:, :c] + p[:, c : 2 * c] + p[:, 2 * c :]


def _normalize_body(c_ref, o_ref):
    c = c_ref[0]
    c = c - jnp.mean(c, axis=0, keepdims=True)
    nrm = jnp.sqrt(jnp.sum(c * c, axis=-1))
    o_ref[0] = c / (jnp.max(nrm) + 1e-8)


def _normalize(coords):
    b, n, _ = coords.shape
    return pl.pallas_call(
        _normalize_body,
        grid=(b,),
        in_specs=[pl.BlockSpec((1, n, 3), lambda i: (i, 0, 0))],
        out_specs=pl.BlockSpec((1, n, 3), lambda i: (i, 0, 0)),
        out_shape=jax.ShapeDtypeStruct((b, n, 3), F32),
    )(coords)


def _shot_body(c3_ref, ct_ref, cb_ref, sig_ref, d2_ref, dx_ref, dy_ref,
               dz_ref, dd_ref, n):
    i = pl.program_id(1)
    c3 = c3_ref[0]
    ct = ct_ref[0]
    cb = cb_ref[0]
    colio = lax.broadcasted_iota(I32, (ROWS, n), 1)
    _build_d2(d2_ref, cb, ct, i * ROWS, n)
    kio = lax.broadcasted_iota(I32, (1, 32), 1)

    def step(k, _):
        m, oh = _extract_min(d2_ref, colio, n)
        nbr = _oh_dot3(oh.astype(jnp.bfloat16), c3, 3)
        sel = kio == k
        dx_ref[...] = jnp.where(sel, nbr[:, 0:1] - cb[:, 0:1], dx_ref[...])
        dy_ref[...] = jnp.where(sel, nbr[:, 1:2] - cb[:, 1:2], dy_ref[...])
        dz_ref[...] = jnp.where(sel, nbr[:, 2:3] - cb[:, 2:3], dz_ref[...])
        dd_ref[...] = jnp.where(sel, jnp.sqrt(m + 1e-12), dd_ref[...])
        return 0

    lax.fori_loop(0, 32, step, 0)

    d = dd_ref[...]
    rmax = jnp.max(d, axis=1, keepdims=True) + 1e-8
    x_ = dx_ref[...] + 1e-12
    y = dy_ref[...]
    z = dz_ref[...]
    upper = jnp.where(x_ > 0, jnp.where(y < x_, 4, 5),
                      jnp.where(y > -x_, 6, 7))
    lower = jnp.where(x_ < 0, jnp.where(-y < -x_, 0, 1),
                      jnp.where(-y > x_, 2, 3))
    azb = jnp.where(y >= 0, upper, lower)
    elb = (z / d + 1.0 >= 1.0).astype(I32)
    t = d / rmax
    radb = (t * 2.0 >= 1.0).astype(I32)
    u = t * 11.0
    histb = jnp.zeros_like(azb)
    for j in range(1, 11):
        histb += (u >= j).astype(I32)
    bins = ((azb * 2 + elb) * 2 + radb) * 11 + histb
    binio = lax.broadcasted_iota(I32, (1, 352), 1)
    sig_ref[0] = jnp.zeros((ROWS, 352), F32)
    for k in range(32):
        sig_ref[0] += (bins[:, k : k + 1] == binio).astype(F32)
    sig_ref[0] = sig_ref[0] * np.float32(1.0 / 32.0)


def _shot(c, c3, ct):
    b, n, _ = c.shape
    return pl.pallas_call(
        functools.partial(_shot_body, n=n),
        grid=(b, n // ROWS),
        in_specs=[
            pl.BlockSpec((1, n, 9), lambda bi, i: (bi, 0, 0)),
            pl.BlockSpec((1, 3, n), lambda bi, i: (bi, 0, 0)),
            pl.BlockSpec((1, ROWS, 3), lambda bi, i: (bi, i, 0)),
        ],
        out_specs=pl.BlockSpec((1, ROWS, 352), lambda bi, i: (bi, i, 0)),
        out_shape=jax.ShapeDtypeStruct((b, n, 352), F32),
        scratch_shapes=[
            pltpu.VMEM((ROWS, n), F32),
            pltpu.VMEM((ROWS, 32), F32),
            pltpu.VMEM((ROWS, 32), F32),
            pltpu.VMEM((ROWS, 32), F32),
            pltpu.VMEM((ROWS, 32), F32),
        ],
    )(c3, ct, c)


def _bary_body(c3_ref, ct_ref, cb_ref, sig_ref, w_ref, g_ref,
               d2_ref, l_ref, n, radius, cdim, stage0):
    i = pl.program_id(1)
    c3 = c3_ref[0]
    ct = ct_ref[0]
    cb = cb_ref[0]
    colio = lax.broadcasted_iota(I32, (ROWS, n), 1)
    _build_d2(d2_ref, cb, ct, i * ROWS, n)

    tio = lax.broadcasted_iota(I32, (1, 40), 1)
    rad_i = (tio // 8).astype(F32)
    ang_i = (tio % 8).astype(F32)
    rr2 = np.float32(radius / 5.0) * (rad_i + 1.0)
    ppv = np.float32(2.0 * np.pi / 8.0) * (ang_i + 0.5) - np.float32(np.pi)
    cpp = jnp.cos(ppv)
    spp = jnp.sin(ppv)
    inv_r2 = np.float32(1.0 / (radius * radius))

    def step(k, _):
        m, oh = _extract_min(d2_ref, colio, n)
        ohb = oh.astype(jnp.bfloat16)
        if stage0:
            g_ref[0, k] = jnp.dot(oh.astype(F32), sig_ref[0],
                                  precision=lax.Precision.DEFAULT)
        else:
            g_ref[0, k] = _oh_dot3(ohb, sig_ref[0], cdim)
        nbr = _oh_dot3(ohb, c3, 3)
        relx = nbr[:, 0:1] - cb[:, 0:1] + 1e-12
        rely = nbr[:, 1:2] - cb[:, 1:2]
        hyp = jnp.maximum(jnp.sqrt(relx * relx + rely * rely), 1e-30)
        cost = relx / hyp
        sint = rely / hyp
        d = jnp.sqrt(m + 1e-12)
        dr = d - rr2
        logit = -LAM * (dr * dr * inv_r2 + (1.0 - (cost * cpp + sint * spp)))
        l_ref[k] = logit[None]
        return 0

    lax.fori_loop(0, 10, step, 0)

    mx = l_ref[0]
    for k in range(1, 10):
        mx = jnp.maximum(mx, l_ref[k])
    s = jnp.zeros((1, ROWS, 40), F32)
    for k in range(10):
        e = jnp.exp(l_ref[k] - mx)
        s += e
        w_ref[0, k] = e[0]
    inv_s = (1.0 / s)[0]
    for k in range(10):
        w_ref[0, k] = w_ref[0, k] * inv_s


def _bary(c3, ct, cb, sig, radius, cdim, stage0):
    b, n, _ = cb.shape
    sdim = sig.shape[-1]
    return pl.pallas_call(
        functools.partial(_bary_body, n=n, radius=radius, cdim=cdim,
                          stage0=stage0),
        grid=(b, n // ROWS),
        in_specs=[
            pl.BlockSpec((1, n, 9), lambda bi, i: (bi, 0, 0)),
            pl.BlockSpec((1, 3, n), lambda bi, i: (bi, 0, 0)),
            pl.BlockSpec((1, ROWS, 3), lambda bi, i: (bi, i, 0)),
            pl.BlockSpec((1, n, sdim), lambda bi, i: (bi, 0, 0)),
        ],
        out_specs=[
            pl.BlockSpec((1, 10, ROWS, 40), lambda bi, i: (bi, 0, i, 0)),
            pl.BlockSpec((1, 10, ROWS, cdim), lambda bi, i: (bi, 0, i, 0)),
        ],
        out_shape=[
            jax.ShapeDtypeStruct((b, 10, n, 40), F32),
            jax.ShapeDtypeStruct((b, 10, n, cdim), F32),
        ],
        scratch_shapes=[
            pltpu.VMEM((ROWS, n), F32),
            pltpu.VMEM((10, 1, ROWS, 40), F32),
        ],
    )(c3, ct, cb, sig)


def _resnet_body(g_ref, w_ref, sig_ref, wt_ref, r_ref, b_ref, out_ref,
                 o3_ref, acc_ref):
    sig = sig_ref[0]
    acc_ref[...] = jnp.dot(sig, r_ref[...], precision=HI) + b_ref[...]
    for t in range(40):
        interp = w_ref[0, 0, :, t : t + 1] * g_ref[0, 0]
        for k in range(1, 10):
            interp += w_ref[0, k, :, t : t + 1] * g_ref[0, k]
        acc_ref[...] += jnp.dot(interp, wt_ref[t], precision=HI)
    relu = jnp.maximum(acc_ref[...], 0.0)
    out_ref[0] = relu
    o3_ref[0] = _split3cat(relu)


def _resnet(g, w, sig, wt, r, bias):
    b, n, cdim = sig.shape
    f = wt.shape[-1]
    return pl.pallas_call(
        _resnet_body,
        grid=(b, n // ROWS),
        in_specs=[
            pl.BlockSpec((1, 10, ROWS, cdim), lambda bi, i: (bi, 0, i, 0)),
            pl.BlockSpec((1, 10, ROWS, 40), lambda bi, i: (bi, 0, i, 0)),
            pl.BlockSpec((1, ROWS, cdim), lambda bi, i: (bi, i, 0)),
            pl.BlockSpec((40, cdim, f), lambda bi, i: (0, 0, 0)),
            pl.BlockSpec((cdim, f), lambda bi, i: (0, 0)),
            pl.BlockSpec((1, f), lambda bi, i: (0, 0)),
        ],
        out_specs=[
            pl.BlockSpec((1, ROWS, f), lambda bi, i: (bi, i, 0)),
            pl.BlockSpec((1, ROWS, 3 * f), lambda bi, i: (bi, i, 0)),
        ],
        out_shape=[
            jax.ShapeDtypeStruct((b, n, f), F32),
            jax.ShapeDtypeStruct((b, n, 3 * f), jnp.bfloat16),
        ],
        scratch_shapes=[pltpu.VMEM((ROWS, f), F32)],
    )(g, w, sig, wt, r, bias.reshape(1, f))


def _gravity_body(c3_ref, ct_ref, cb_ref, out_ref, d2_ref, n, rstride):
    i = pl.program_id(1)
    c3 = c3_ref[0]
    ct = ct_ref[0]
    cb = cb_ref[0]
    colio = lax.broadcasted_iota(I32, (ROWS, n), 1)
    _build_d2(d2_ref, cb, ct, i * ROWS, n, rstride)

    def step(k, _):
        d2c = d2_ref[...]
        m = jnp.min(d2c, axis=1, keepdims=True)
        am = jnp.min(jnp.where(d2c == m, colio, n), axis=1, keepdims=True)
        d2_ref[...] = jnp.where(colio == am, BIG, d2c)
        return 0

    lax.fori_loop(0, 16, step, 0)
    rio = (i * ROWS + lax.broadcasted_iota(I32, (ROWS, n), 0)) * rstride
    ohb = ((d2_ref[...] >= BIG) & (colio != rio)).astype(jnp.bfloat16)
    nm = _oh_dot3(ohb, c3, 3) * np.float32(1.0 / 16.0)
    out_ref[0] = cb + THIRD * (nm - cb)


def _gravity(c3, ct, cb, rstride=1):
    b, n, _ = ct.shape[0], ct.shape[2], 3
    nr = cb.shape[1]
    return pl.pallas_call(
        functools.partial(_gravity_body, n=n, rstride=rstride),
        grid=(b, nr // ROWS),
        in_specs=[
            pl.BlockSpec((1, n, 9), lambda bi, i: (bi, 0, 0)),
            pl.BlockSpec((1, 3, n), lambda bi, i: (bi, 0, 0)),
            pl.BlockSpec((1, ROWS, 3), lambda bi, i: (bi, i, 0)),
        ],
        out_specs=pl.BlockSpec((1, ROWS, 3), lambda bi, i: (bi, i, 0)),
        out_shape=jax.ShapeDtypeStruct((b, nr, 3), F32),
        scratch_shapes=[
            pltpu.VMEM((ROWS, n), F32),
        ],
    )(c3, ct, cb)


def _clf_body(sig_ref, w_ref, b_ref, out_ref):
    msig = jnp.mean(sig_ref[0], axis=0, keepdims=True)
    out_ref[0] = jnp.dot(msig, w_ref[...], precision=HI) + b_ref[...]


def _clf(sig, clf_w, clf_b):
    b, n, cdim = sig.shape
    out = pl.pallas_call(
        _clf_body,
        grid=(b,),
        in_specs=[
            pl.BlockSpec((1, n, cdim), lambda bi: (bi, 0, 0)),
            pl.BlockSpec((cdim, 40), lambda bi: (0, 0)),
            pl.BlockSpec((1, 40), lambda bi: (0, 0)),
        ],
        out_specs=pl.BlockSpec((1, 1, 40), lambda bi: (bi, 0, 0)),
        out_shape=jax.ShapeDtypeStruct((b, 1, 40), F32),
    )(sig, clf_w, clf_b.reshape(1, 40))
    return out.reshape(b, 40)


def kernel(coordinates, W0, b0, R0, W1, b1, R1, W2, b2, R2, clf_w, clf_b):
    Ws = [W0, W1, W2]
    bs = [b0, b1, b2]
    Rs = [R0, R1, R2]
    c = _normalize(coordinates)
    ct = c.transpose(0, 2, 1)
    c3 = _split3cat(c)
    sig = _shot(c, c3, ct)
    sig3 = None
    for i in range(3):
        radius = 0.05 * (1.0 + 0.25 * i)
        w, g = _bary(c3, ct, c, sig if i == 0 else sig3, radius,
                     CIN[i], stage0=(i == 0))
        sig, sig3 = _resnet(g, w, sig, Ws[i], Rs[i], bs[i])
        for s in range(3):
            c3 = _split3cat(c)
            if s < 2:
                c = _gravity(c3, ct, c)
            else:
                c = _gravity(c3, ct, c[:, ::2], rstride=2)
            ct = c.transpose(0, 2, 1)
        c3 = _split3cat(c)
        sig = sig[:, ::2]
        sig3 = sig3[:, ::2]
    return _clf(sig, clf_w, clf_b)
